# baseline (speedup 1.0000x reference)
import functools

import jax
import jax.numpy as jnp
import numpy as np
from jax.experimental import pallas as pl
from jax.experimental.pallas import tpu as pltpu

B = 10000
S = 8
N_IN = 256
N_H = 64

B_BLK = 2000
SUB = 16
N_SUB = B_BLK // SUB
GRID = B // B_BLK

_SEL = np.zeros((48, 128), dtype=np.float32)
for _i in range(16):
    _SEL[_i, _i * 8 + 1:_i * 8 + 8] = 1.0 / 7.0
    _SEL[16 + _i, _i * 8 + 0] = 1.0
    _SEL[32 + _i, _i * 8 + 1] = 1.0

_ONES2 = np.zeros((128, 2), dtype=np.float32)
_ONES2[0:64, 0] = 1.0
_ONES2[64:128, 1] = 1.0


def _body(seq_ref, adj_ref, wcp_ref, sel_ref, wkbd_ref, bias_ref,
          slope_ref, mask8_ref, ones2_ref, bk_ref, ret1_ref, ret2_ref,
          zscr, chscr, carry_ref, zrow0_ref):
    i = pl.program_id(0)
    base = i * B_BLK
    x = seq_ref[...].reshape(B_BLK * S, N_IN).astype(jnp.bfloat16)
    fts = jnp.dot(x, wcp_ref[...],
                  preferred_element_type=jnp.float32).astype(jnp.bfloat16)
    sel = sel_ref[...]
    bias = bias_ref[...]
    slope = slope_ref[...]
    mask8 = mask8_ref[...].reshape(1, 1, 128)
    for j in range(N_SUB):
        rows = slice(j * 128, (j + 1) * 128)
        d16 = adj_ref[j * SUB:(j + 1) * SUB, :]
        v = jnp.broadcast_to(d16[:, None, :], (SUB, S, 128))
        v = pltpu.roll(v, 8, 2, stride=8, stride_axis=1)
        v = v * mask8
        v = pltpu.roll(v, 64, 2)
        v = pltpu.roll(v, 0, 2, stride=8, stride_axis=0)
        bd = v.reshape(128, 128)
        h = jnp.dot(bd, fts[rows, :], preferred_element_type=jnp.float32)
        y = h.astype(jnp.bfloat16) + bias
        hact = jnp.where(y >= 0, y, slope * y)
        r = jnp.dot(sel, hact, preferred_element_type=jnp.float32)
        rb = r.astype(jnp.bfloat16)
        o = slice(j * SUB, (j + 1) * SUB)
        zscr[o, :] = rb[16:32, :]
        chscr[o, :] = jnp.concatenate([rb[0:16, 0:64], rb[32:48, 64:128]],
                                      axis=1)

    zb = jnp.dot(zscr[...], wkbd_ref[...],
                 preferred_element_type=jnp.float32).astype(jnp.bfloat16)
    ch = chscr[...]
    ones2 = ones2_ref[...]
    bkc = bk_ref[0, 0]
    bkp = bk_ref[0, 1]

    @pl.when(i == 0)
    def _():
        zrow0_ref[0:1, :] = zb[0:1, :]

    st0 = jnp.dot(zb * ch, ones2, preferred_element_type=jnp.float32)
    chsh = jnp.concatenate([carry_ref[0:1, :], ch[0:B_BLK - 1, :]], axis=0)
    st1 = jnp.dot(zb * chsh, ones2, preferred_element_type=jnp.float32)
    carry_ref[0:1, :] = ch[B_BLK - 1:B_BLK, :]
    ret1_ref[pl.ds(base, B_BLK), :] = st0[:, 0:1] + bkc
    ret2_ref[pl.ds(base, B_BLK), :] = st0[:, 1:2] + bkp
    ret1_ref[pl.ds(B + base, B_BLK), :] = st1[:, 0:1] + bkc
    ret2_ref[pl.ds(B + base, B_BLK), :] = st1[:, 1:2] + bkp

    @pl.when(i == GRID - 1)
    def _():
        pz = zrow0_ref[0:1, :] * chscr[B_BLK - 2:B_BLK - 1, :]
        sw = jnp.dot(pz, ones2, preferred_element_type=jnp.float32)
        ret1_ref[B:B + 1, :] = sw[:, 0:1] + bkc
        ret2_ref[B:B + 1, :] = sw[:, 1:2] + bkp


@functools.partial(jax.jit, static_argnames=("interpret",))
def _run(seq1, adj, Wc, bc, a_c, Wp, bp, a_p, Wk_c, bk_c, Wk_p, bk_p,
         interpret=False):
    adjd = jnp.pad(adj.astype(jnp.bfloat16).reshape(B, S * S),
                   ((0, 0), (0, 128 - S * S)))
    mask8 = jnp.asarray(((np.arange(128) >= 64) & (np.arange(128) < 72))
                        .astype(np.float32))[None, :].astype(jnp.bfloat16)

    wcp = jnp.concatenate([Wc.T, Wp.T], axis=1).astype(jnp.bfloat16)
    wkbd = jnp.zeros((128, 128), jnp.float32)
    wkbd = wkbd.at[0:64, 0:64].set(Wk_c).at[64:128, 64:128].set(Wk_p)
    wkbd = wkbd.astype(jnp.bfloat16)
    bias = jnp.concatenate([bc, bp])[None, :].astype(jnp.bfloat16)
    slope = jnp.concatenate([jnp.broadcast_to(a_c, (64,)),
                             jnp.broadcast_to(a_p, (64,))]
                            )[None, :].astype(jnp.bfloat16)
    bk = jnp.stack([bk_c[0], bk_p[0]])[None, :]
    sel = jnp.asarray(_SEL).astype(jnp.bfloat16)
    ones2 = jnp.asarray(_ONES2).astype(jnp.bfloat16)

    ret1, ret2 = pl.pallas_call(
        _body,
        grid=(GRID,),
        in_specs=[
            pl.BlockSpec((B_BLK, S, N_IN), lambda i: (i, 0, 0)),
            pl.BlockSpec((B_BLK, 128), lambda i: (i, 0)),
            pl.BlockSpec((N_IN, 128), lambda i: (0, 0)),
            pl.BlockSpec((48, 128), lambda i: (0, 0)),
            pl.BlockSpec((128, 128), lambda i: (0, 0)),
            pl.BlockSpec((1, 128), lambda i: (0, 0)),
            pl.BlockSpec((1, 128), lambda i: (0, 0)),
            pl.BlockSpec((1, 128), lambda i: (0, 0)),
            pl.BlockSpec((128, 2), lambda i: (0, 0)),
            pl.BlockSpec((1, 2), lambda i: (0, 0)),
        ],
        out_specs=[
            pl.BlockSpec((2 * B, 1), lambda i: (0, 0)),
            pl.BlockSpec((2 * B, 1), lambda i: (0, 0)),
        ],
        out_shape=[
            jax.ShapeDtypeStruct((2 * B, 1), jnp.float32),
            jax.ShapeDtypeStruct((2 * B, 1), jnp.float32),
        ],
        scratch_shapes=[
            pltpu.VMEM((B_BLK, 128), jnp.bfloat16),
            pltpu.VMEM((B_BLK, 128), jnp.bfloat16),
            pltpu.VMEM((8, 128), jnp.bfloat16),
            pltpu.VMEM((8, 128), jnp.bfloat16),
        ],
        interpret=interpret,
    )(seq1, adjd, wcp, sel, wkbd, bias, slope, mask8, ones2, bk)
    return ret1, ret2


def kernel(seq1, adj, Wc, bc, a_c, Wp, bp, a_p, Wk_c, bk_c, Wk_p, bk_p):
    return _run(seq1, adj, Wc, bc, a_c, Wp, bp, a_p,
                Wk_c, bk_c, Wk_p, bk_p)


# speedup vs baseline: 1.0078x; 1.0078x over previous
import functools

import jax
import jax.numpy as jnp
import numpy as np
from jax.experimental import pallas as pl
from jax.experimental.pallas import tpu as pltpu

B = 10000
S = 8
N_IN = 256
N_H = 64

B_BLK = 2000
SUB = 16
N_SUB = B_BLK // SUB
GRID = B // B_BLK

_SEL = np.zeros((48, 128), dtype=np.float32)
for _i in range(16):
    _SEL[_i, _i * 8 + 1:_i * 8 + 8] = 1.0 / 7.0
    _SEL[16 + _i, _i * 8 + 0] = 1.0
    _SEL[32 + _i, _i * 8 + 1] = 1.0

_ONES2 = np.zeros((128, 2), dtype=np.float32)
_ONES2[0:64, 0] = 1.0
_ONES2[64:128, 1] = 1.0


def _body(seq_ref, adj_ref, wcp_ref, sel_ref, wkbd_ref, bias_ref,
          slope_ref, mask8_ref, ones2_ref, bk_ref, ret1_ref, ret2_ref,
          zscr, chscr, carry_ref, zrow0_ref):
    i = pl.program_id(0)
    base = i * B_BLK
    x = seq_ref[...].reshape(B_BLK * S, N_IN).astype(jnp.bfloat16)
    fts = jnp.dot(x, wcp_ref[...],
                  preferred_element_type=jnp.float32).astype(jnp.bfloat16)
    sel = sel_ref[...]
    bias = bias_ref[...]
    slope = slope_ref[...]
    mask8 = mask8_ref[...].reshape(1, 1, 128)
    for j in range(N_SUB):
        rows = slice(j * 128, (j + 1) * 128)
        d16 = adj_ref[j * SUB:(j + 1) * SUB, :]
        u = jnp.broadcast_to(d16[:, None, :], (SUB, S, 64))
        v = jnp.pad(u, ((0, 0), (0, 0), (0, 64)))
        v = pltpu.roll(v, 8, 2, stride=8, stride_axis=1)
        v = v * mask8
        v = pltpu.roll(v, 64, 2)
        v = pltpu.roll(v, 0, 2, stride=8, stride_axis=0)
        bd = v.reshape(128, 128)
        h = jnp.dot(bd, fts[rows, :], preferred_element_type=jnp.float32)
        y = h.astype(jnp.bfloat16) + bias
        hact = jnp.where(y >= 0, y, slope * y)
        r = jnp.dot(sel, hact, preferred_element_type=jnp.float32)
        rb = r.astype(jnp.bfloat16)
        o = slice(j * SUB, (j + 1) * SUB)
        zscr[o, :] = rb[16:32, :]
        chscr[o, :] = jnp.concatenate([rb[0:16, 0:64], rb[32:48, 64:128]],
                                      axis=1)

    zb = jnp.dot(zscr[...], wkbd_ref[...],
                 preferred_element_type=jnp.float32).astype(jnp.bfloat16)
    ch = chscr[...]
    ones2 = ones2_ref[...]
    bkc = bk_ref[0, 0]
    bkp = bk_ref[0, 1]

    @pl.when(i == 0)
    def _():
        zrow0_ref[0:1, :] = zb[0:1, :]

    st0 = jnp.dot(zb * ch, ones2, preferred_element_type=jnp.float32)
    chsh = jnp.concatenate([carry_ref[0:1, :], ch[0:B_BLK - 1, :]], axis=0)
    st1 = jnp.dot(zb * chsh, ones2, preferred_element_type=jnp.float32)
    carry_ref[0:1, :] = ch[B_BLK - 1:B_BLK, :]
    ret1_ref[pl.ds(base, B_BLK), :] = st0[:, 0:1] + bkc
    ret2_ref[pl.ds(base, B_BLK), :] = st0[:, 1:2] + bkp
    ret1_ref[pl.ds(B + base, B_BLK), :] = st1[:, 0:1] + bkc
    ret2_ref[pl.ds(B + base, B_BLK), :] = st1[:, 1:2] + bkp

    @pl.when(i == GRID - 1)
    def _():
        pz = zrow0_ref[0:1, :] * chscr[B_BLK - 2:B_BLK - 1, :]
        sw = jnp.dot(pz, ones2, preferred_element_type=jnp.float32)
        ret1_ref[B:B + 1, :] = sw[:, 0:1] + bkc
        ret2_ref[B:B + 1, :] = sw[:, 1:2] + bkp


@functools.partial(jax.jit, static_argnames=("interpret",))
def _run(seq1, adj, Wc, bc, a_c, Wp, bp, a_p, Wk_c, bk_c, Wk_p, bk_p,
         interpret=False):
    adjd = adj.astype(jnp.bfloat16).reshape(B, S * S)
    mask8 = jnp.asarray(((np.arange(128) >= 64) & (np.arange(128) < 72))
                        .astype(np.float32))[None, :].astype(jnp.bfloat16)

    wcp = jnp.concatenate([Wc.T, Wp.T], axis=1).astype(jnp.bfloat16)
    wkbd = jnp.zeros((128, 128), jnp.float32)
    wkbd = wkbd.at[0:64, 0:64].set(Wk_c).at[64:128, 64:128].set(Wk_p)
    wkbd = wkbd.astype(jnp.bfloat16)
    bias = jnp.concatenate([bc, bp])[None, :].astype(jnp.bfloat16)
    slope = jnp.concatenate([jnp.broadcast_to(a_c, (64,)),
                             jnp.broadcast_to(a_p, (64,))]
                            )[None, :].astype(jnp.bfloat16)
    bk = jnp.stack([bk_c[0], bk_p[0]])[None, :]
    sel = jnp.asarray(_SEL).astype(jnp.bfloat16)
    ones2 = jnp.asarray(_ONES2).astype(jnp.bfloat16)

    ret1, ret2 = pl.pallas_call(
        _body,
        grid=(GRID,),
        in_specs=[
            pl.BlockSpec((B_BLK, S, N_IN), lambda i: (i, 0, 0)),
            pl.BlockSpec((B_BLK, S * S), lambda i: (i, 0)),
            pl.BlockSpec((N_IN, 128), lambda i: (0, 0)),
            pl.BlockSpec((48, 128), lambda i: (0, 0)),
            pl.BlockSpec((128, 128), lambda i: (0, 0)),
            pl.BlockSpec((1, 128), lambda i: (0, 0)),
            pl.BlockSpec((1, 128), lambda i: (0, 0)),
            pl.BlockSpec((1, 128), lambda i: (0, 0)),
            pl.BlockSpec((128, 2), lambda i: (0, 0)),
            pl.BlockSpec((1, 2), lambda i: (0, 0)),
        ],
        out_specs=[
            pl.BlockSpec((2 * B, 1), lambda i: (0, 0)),
            pl.BlockSpec((2 * B, 1), lambda i: (0, 0)),
        ],
        out_shape=[
            jax.ShapeDtypeStruct((2 * B, 1), jnp.float32),
            jax.ShapeDtypeStruct((2 * B, 1), jnp.float32),
        ],
        scratch_shapes=[
            pltpu.VMEM((B_BLK, 128), jnp.bfloat16),
            pltpu.VMEM((B_BLK, 128), jnp.bfloat16),
            pltpu.VMEM((8, 128), jnp.bfloat16),
            pltpu.VMEM((8, 128), jnp.bfloat16),
        ],
        interpret=interpret,
    )(seq1, adjd, wcp, sel, wkbd, bias, slope, mask8, ones2, bk)
    return ret1, ret2


def kernel(seq1, adj, Wc, bc, a_c, Wp, bp, a_p, Wk_c, bk_c, Wk_p, bk_p):
    return _run(seq1, adj, Wc, bc, a_c, Wp, bp, a_p,
                Wk_c, bk_c, Wk_p, bk_p)
